# baseline (speedup 1.0000x reference)
import functools

import jax
import jax.numpy as jnp
from jax import lax
from jax.experimental import pallas as pl
from jax.experimental.pallas import tpu as pltpu
from jax.experimental.pallas import tpu_sc as plsc

N = 10000
E = 320000
NUM_CLASSES = 40
FEATURE_DIM = 128
FEATURE_HIDDEN = 64
HIDDEN = 128

NC = 2
NS = 16
NW = NC * NS
CHUNK = 128
NCHUNKS = 80
EPT = NCHUNKS * CHUNK
EPAD = NW * EPT
NPAD = 10240
ROWS_PT = NPAD // NS
NB = 4

_mesh = plsc.VectorSubcoreMesh(core_axis_name="c", subcore_axis_name="s")


def _make_deg_kernel():
    @functools.partial(
        pl.kernel,
        mesh=_mesh,
        compiler_params=pltpu.CompilerParams(use_tc_tiling_on_sc=False),
        out_type=jax.ShapeDtypeStruct((NC, NPAD), jnp.float32),
        scratch_types=[
            pltpu.VMEM((NCHUNKS, CHUNK), jnp.int32),
            pltpu.VMEM((CHUNK,), jnp.float32),
            pltpu.VMEM_SHARED((NPAD,), jnp.float32),
            pltpu.SemaphoreType.DMA,
        ],
    )
    def deg_kernel(dst_hbm, ones_hbm, zeros_hbm, out_hbm, dsts_v, ones_v,
                   acc_sh, sem):
        c = lax.axis_index("c")
        s = lax.axis_index("s")
        wid = c * NS + s
        rbase = pl.multiple_of(s * ROWS_PT, 8)
        pltpu.sync_copy(zeros_hbm.at[pl.ds(rbase, ROWS_PT)],
                        acc_sh.at[pl.ds(rbase, ROWS_PT)])
        pltpu.sync_copy(dst_hbm.at[wid], dsts_v)
        pltpu.sync_copy(ones_hbm, ones_v)
        plsc.subcore_barrier()

        def body(j, carry):
            pltpu.async_copy(ones_v, acc_sh.at[dsts_v.at[j]], sem, add=True)
            return carry

        lax.fori_loop(0, NCHUNKS, body, 0)

        def drain(j, carry):
            pltpu.make_async_copy(ones_v, acc_sh.at[dsts_v.at[0]], sem).wait()
            return carry

        lax.fori_loop(0, NCHUNKS, drain, 0)
        plsc.subcore_barrier()
        pltpu.sync_copy(acc_sh.at[pl.ds(rbase, ROWS_PT)],
                        out_hbm.at[c, pl.ds(rbase, ROWS_PT)])

    return deg_kernel


def _make_spmm_kernel(D):
    HC = NCHUNKS // 2

    @functools.partial(
        pl.kernel,
        mesh=_mesh,
        compiler_params=pltpu.CompilerParams(use_tc_tiling_on_sc=False),
        out_type=jax.ShapeDtypeStruct((NC, NPAD, D), jnp.float32),
        scratch_types=[
            pltpu.VMEM((HC, CHUNK), jnp.int32),
            pltpu.VMEM((HC, CHUNK), jnp.int32),
            pltpu.VMEM((2, CHUNK, D), jnp.float32),
            pltpu.VMEM_SHARED((NPAD, D), jnp.float32),
            pltpu.SemaphoreType.DMA,
            pltpu.SemaphoreType.DMA,
        ],
    )
    def spmm_kernel(src_hbm, dst_hbm, g_hbm, zeros_hbm, out_hbm,
                    srcs_v, dsts_v, rows_v, acc_sh, sem0, sem1):
        sems = (sem0, sem1)
        c = lax.axis_index("c")
        s = lax.axis_index("s")
        wid = c * NS + s
        rbase = pl.multiple_of(s * ROWS_PT, 8)
        pltpu.sync_copy(zeros_hbm.at[pl.ds(rbase, ROWS_PT)],
                        acc_sh.at[pl.ds(rbase, ROWS_PT)])
        plsc.subcore_barrier()

        for h in range(2):
            pltpu.sync_copy(src_hbm.at[wid, pl.ds(h * HC, HC)], srcs_v)
            pltpu.sync_copy(dst_hbm.at[wid, pl.ds(h * HC, HC)], dsts_v)
            pltpu.async_copy(g_hbm.at[srcs_v.at[0]], rows_v.at[0], sems[0])

            def group(g2, carry):
                for b in range(2):
                    j = g2 * 2 + b
                    pltpu.make_async_copy(g_hbm.at[srcs_v.at[0]],
                                          rows_v.at[b], sems[b]).wait()
                    jn = jnp.where(j + 1 < HC, j + 1, 0)
                    pltpu.async_copy(g_hbm.at[srcs_v.at[jn]],
                                     rows_v.at[b ^ 1], sems[b ^ 1])
                    pltpu.sync_copy(rows_v.at[b], acc_sh.at[dsts_v.at[j]],
                                    add=True)
                return carry

            lax.fori_loop(0, HC // 2, group, 0)
            pltpu.make_async_copy(g_hbm.at[srcs_v.at[0]], rows_v.at[0],
                                  sems[0]).wait()

        plsc.subcore_barrier()
        pltpu.sync_copy(acc_sh.at[pl.ds(rbase, ROWS_PT)],
                        out_hbm.at[c, pl.ds(rbase, ROWS_PT)])

    return spmm_kernel


_deg_kernel = _make_deg_kernel()
_spmm128 = _make_spmm_kernel(HIDDEN)
_spmm64 = _make_spmm_kernel(FEATURE_HIDDEN)

BR = 512
GRID = NPAD // BR


def _tc1_body(logits_ref, feat_ref, wpt_ref, bp_ref, w1at_ref, w1bt_ref,
              d0_ref, d1_ref, g1_ref, dinv_ref):
    fp = jnp.dot(feat_ref[...], wpt_ref[...],
                 preferred_element_type=jnp.float32) + bp_ref[...]
    h1 = (jnp.dot(logits_ref[...], w1at_ref[...],
                  preferred_element_type=jnp.float32)
          + jnp.dot(fp, w1bt_ref[...], preferred_element_type=jnp.float32))
    deg = d0_ref[...] + d1_ref[...] + 1.0
    dinv = lax.rsqrt(deg)
    dinv_ref[...] = dinv
    g1_ref[...] = h1 * dinv


def _tc2_body(s0_ref, s1_ref, g1_ref, dinv_ref, b1_ref, w2t_ref, g2_ref):
    dinv = dinv_ref[...]
    out1 = dinv * (s0_ref[...] + s1_ref[...] + g1_ref[...]) + b1_ref[...]
    x = jnp.maximum(out1, 0.0)
    h2 = jnp.dot(x, w2t_ref[...], preferred_element_type=jnp.float32)
    g2_ref[...] = h2 * dinv


def _tc3_body(t0_ref, t1_ref, g2_ref, dinv_ref, b2_ref, out_ref):
    out_ref[...] = (dinv_ref[...] * (t0_ref[...] + t1_ref[...] + g2_ref[...])
                    + b2_ref[...])


def _row_spec(cols):
    return pl.BlockSpec((BR, cols), lambda r: (r, 0))


def _full_spec(rows, cols):
    return pl.BlockSpec((rows, cols), lambda r: (0, 0))


def kernel(logits, features, edge_index, Wp, bp, W1, b1, W2, b2):
    src = edge_index[0]
    dst = edge_index[1]
    pad_e = EPAD - E
    srcp = jnp.concatenate([src, jnp.full((pad_e,), N, jnp.int32)])
    dstp = jnp.concatenate([dst, jnp.full((pad_e,), N, jnp.int32)])
    srcp = srcp.reshape(NW, NCHUNKS, CHUNK)
    dstp = dstp.reshape(NW, NCHUNKS, CHUNK)

    zeros128 = jnp.zeros((NPAD, HIDDEN), jnp.float32)
    zeros64 = jnp.zeros((NPAD, FEATURE_HIDDEN), jnp.float32)
    zeros1 = jnp.zeros((NPAD,), jnp.float32)
    ones_c = jnp.ones((CHUNK,), jnp.float32)

    logits_p = jnp.zeros((NPAD, NUM_CLASSES), jnp.float32).at[:N].set(logits)
    features_p = jnp.zeros((NPAD, FEATURE_DIM), jnp.float32).at[:N].set(features)

    WpT = Wp.T
    W1aT = W1[:, :NUM_CLASSES].T
    W1bT = W1[:, NUM_CLASSES:].T
    W2Tp = jnp.zeros((HIDDEN, FEATURE_HIDDEN), jnp.float32).at[:, :NUM_CLASSES].set(W2.T)
    b2p = jnp.zeros((FEATURE_HIDDEN,), jnp.float32).at[:NUM_CLASSES].set(b2)

    dpart = _deg_kernel(dstp, ones_c, zeros1)
    d0 = dpart[0].reshape(NPAD, 1)
    d1 = dpart[1].reshape(NPAD, 1)

    g1, dinv = pl.pallas_call(
        _tc1_body,
        grid=(GRID,),
        in_specs=[
            _row_spec(NUM_CLASSES),
            _row_spec(FEATURE_DIM),
            _full_spec(FEATURE_DIM, FEATURE_HIDDEN),
            _full_spec(1, FEATURE_HIDDEN),
            _full_spec(NUM_CLASSES, HIDDEN),
            _full_spec(FEATURE_HIDDEN, HIDDEN),
            _row_spec(1),
            _row_spec(1),
        ],
        out_specs=[_row_spec(HIDDEN), _row_spec(1)],
        out_shape=[
            jax.ShapeDtypeStruct((NPAD, HIDDEN), jnp.float32),
            jax.ShapeDtypeStruct((NPAD, 1), jnp.float32),
        ],
    )(logits_p, features_p, WpT, bp.reshape(1, -1), W1aT, W1bT, d0, d1)

    spart = _spmm128(srcp, dstp, g1, zeros128)

    g2 = pl.pallas_call(
        _tc2_body,
        grid=(GRID,),
        in_specs=[
            _row_spec(HIDDEN),
            _row_spec(HIDDEN),
            _row_spec(HIDDEN),
            _row_spec(1),
            _full_spec(1, HIDDEN),
            _full_spec(HIDDEN, FEATURE_HIDDEN),
        ],
        out_specs=_row_spec(FEATURE_HIDDEN),
        out_shape=jax.ShapeDtypeStruct((NPAD, FEATURE_HIDDEN), jnp.float32),
    )(spart[0], spart[1], g1, dinv, b1.reshape(1, -1), W2Tp)

    tpart = _spmm64(srcp, dstp, g2, zeros64)

    out = pl.pallas_call(
        _tc3_body,
        grid=(GRID,),
        in_specs=[
            _row_spec(FEATURE_HIDDEN),
            _row_spec(FEATURE_HIDDEN),
            _row_spec(FEATURE_HIDDEN),
            _row_spec(1),
            _full_spec(1, FEATURE_HIDDEN),
        ],
        out_specs=_row_spec(FEATURE_HIDDEN),
        out_shape=jax.ShapeDtypeStruct((NPAD, FEATURE_HIDDEN), jnp.float32),
    )(tpart[0], tpart[1], g2, dinv, b2p.reshape(1, -1))

    return out[:N, :NUM_CLASSES]


# speedup vs baseline: 1.8054x; 1.8054x over previous
import functools

import jax
import jax.numpy as jnp
from jax import lax
from jax.experimental import pallas as pl
from jax.experimental.pallas import tpu as pltpu
from jax.experimental.pallas import tpu_sc as plsc

N = 10000
E = 320000
NUM_CLASSES = 40
FEATURE_DIM = 128
FEATURE_HIDDEN = 64
HIDDEN = 128

NC = 2
NS = 16
CHUNK = 128
K0 = 120
K1 = 38
TOT_CHUNKS = NS * (K0 + K1)
EPAD = TOT_CHUNKS * CHUNK
SEG = 40
SEGS0 = (SEG, SEG, SEG)
SEGS1 = (K1,)
NPAD = 10240
ROWS_PT = NPAD // NS

_mesh = plsc.VectorSubcoreMesh(core_axis_name="c", subcore_axis_name="s")


def _make_deg_kernel():
    @functools.partial(
        pl.kernel,
        mesh=_mesh,
        compiler_params=pltpu.CompilerParams(use_tc_tiling_on_sc=False),
        out_type=jax.ShapeDtypeStruct((NC, NPAD), jnp.float32),
        scratch_types=[
            pltpu.VMEM((K0, CHUNK), jnp.int32),
            pltpu.VMEM((CHUNK,), jnp.float32),
            pltpu.VMEM_SHARED((NPAD,), jnp.float32),
            pltpu.SemaphoreType.DMA,
        ],
    )
    def deg_kernel(dst_hbm, ones_hbm, zeros_hbm, out_hbm, dsts_v, ones_v,
                   acc_sh, sem):
        c = lax.axis_index("c")
        s = lax.axis_index("s")
        rbase = pl.multiple_of(s * ROWS_PT, 8)
        pltpu.sync_copy(zeros_hbm.at[pl.ds(rbase, ROWS_PT)],
                        acc_sh.at[pl.ds(rbase, ROWS_PT)])
        pltpu.sync_copy(ones_hbm, ones_v)

        def run(cstart, k):
            pltpu.sync_copy(dst_hbm.at[pl.ds(cstart, k)],
                            dsts_v.at[pl.ds(0, k)])
            plsc.subcore_barrier()

            def body(j, carry):
                pltpu.async_copy(ones_v, acc_sh.at[dsts_v.at[j]], sem,
                                 add=True)
                return carry

            lax.fori_loop(0, k, body, 0)

            def drain(j, carry):
                pltpu.make_async_copy(ones_v, acc_sh.at[dsts_v.at[0]],
                                      sem).wait()
                return carry

            lax.fori_loop(0, k, drain, 0)

        @pl.when(c == 0)
        def _():
            run(s * K0, K0)

        @pl.when(c == 1)
        def _():
            run(NS * K0 + s * K1, K1)

        plsc.subcore_barrier()
        pltpu.sync_copy(acc_sh.at[pl.ds(rbase, ROWS_PT)],
                        out_hbm.at[c, pl.ds(rbase, ROWS_PT)])

    return deg_kernel


def _make_spmm_kernel(D):
    @functools.partial(
        pl.kernel,
        mesh=_mesh,
        compiler_params=pltpu.CompilerParams(use_tc_tiling_on_sc=False),
        out_type=jax.ShapeDtypeStruct((NC, NPAD, D), jnp.float32),
        scratch_types=[
            pltpu.VMEM((SEG, CHUNK), jnp.int32),
            pltpu.VMEM((SEG, CHUNK), jnp.int32),
            pltpu.VMEM((2, CHUNK, D), jnp.float32),
            pltpu.VMEM_SHARED((NPAD, D), jnp.float32),
            pltpu.SemaphoreType.DMA,
            pltpu.SemaphoreType.DMA,
        ],
    )
    def spmm_kernel(src_hbm, dst_hbm, g_hbm, zeros_hbm, out_hbm,
                    srcs_v, dsts_v, rows_v, acc_sh, sem0, sem1):
        sems = (sem0, sem1)
        c = lax.axis_index("c")
        s = lax.axis_index("s")
        rbase = pl.multiple_of(s * ROWS_PT, 8)
        pltpu.sync_copy(zeros_hbm.at[pl.ds(rbase, ROWS_PT)],
                        acc_sh.at[pl.ds(rbase, ROWS_PT)])
        plsc.subcore_barrier()

        def run(cstart, seglens):
            off = 0
            for L in seglens:
                base = cstart + off
                pltpu.sync_copy(src_hbm.at[pl.ds(base, L)],
                                srcs_v.at[pl.ds(0, L)])
                pltpu.sync_copy(dst_hbm.at[pl.ds(base, L)],
                                dsts_v.at[pl.ds(0, L)])
                pltpu.async_copy(g_hbm.at[srcs_v.at[0]], rows_v.at[0],
                                 sems[0])

                def group(g2, carry):
                    for b in range(2):
                        j = g2 * 2 + b
                        pltpu.make_async_copy(g_hbm.at[srcs_v.at[0]],
                                              rows_v.at[b], sems[b]).wait()
                        jn = jnp.where(j + 1 < L, j + 1, 0)
                        pltpu.async_copy(g_hbm.at[srcs_v.at[jn]],
                                         rows_v.at[b ^ 1], sems[b ^ 1])
                        pltpu.sync_copy(rows_v.at[b],
                                        acc_sh.at[dsts_v.at[j]], add=True)
                    return carry

                lax.fori_loop(0, L // 2, group, 0)
                pltpu.make_async_copy(g_hbm.at[srcs_v.at[0]], rows_v.at[0],
                                      sems[0]).wait()
                off += L

        @pl.when(c == 0)
        def _():
            run(s * K0, SEGS0)

        @pl.when(c == 1)
        def _():
            run(NS * K0 + s * K1, SEGS1)

        plsc.subcore_barrier()
        pltpu.sync_copy(acc_sh.at[pl.ds(rbase, ROWS_PT)],
                        out_hbm.at[c, pl.ds(rbase, ROWS_PT)])

    return spmm_kernel


_deg_kernel = _make_deg_kernel()
_spmm128 = _make_spmm_kernel(HIDDEN)
_spmm64 = _make_spmm_kernel(FEATURE_HIDDEN)

BR = 512
GRID = NPAD // BR


def _tc1_body(logits_ref, feat_ref, wpt_ref, bp_ref, w1at_ref, w1bt_ref,
              d0_ref, d1_ref, g1_ref, dinv_ref):
    fp = jnp.dot(feat_ref[...], wpt_ref[...],
                 preferred_element_type=jnp.float32) + bp_ref[...]
    h1 = (jnp.dot(logits_ref[...], w1at_ref[...],
                  preferred_element_type=jnp.float32)
          + jnp.dot(fp, w1bt_ref[...], preferred_element_type=jnp.float32))
    deg = d0_ref[...] + d1_ref[...] + 1.0
    dinv = lax.rsqrt(deg)
    dinv_ref[...] = dinv
    g1_ref[...] = h1 * dinv


def _tc2_body(s0_ref, s1_ref, g1_ref, dinv_ref, b1_ref, w2t_ref, g2_ref):
    dinv = dinv_ref[...]
    out1 = dinv * (s0_ref[...] + s1_ref[...] + g1_ref[...]) + b1_ref[...]
    x = jnp.maximum(out1, 0.0)
    h2 = jnp.dot(x, w2t_ref[...], preferred_element_type=jnp.float32)
    g2_ref[...] = h2 * dinv


def _tc3_body(t0_ref, t1_ref, g2_ref, dinv_ref, b2_ref, out_ref):
    out_ref[...] = (dinv_ref[...] * (t0_ref[...] + t1_ref[...] + g2_ref[...])
                    + b2_ref[...])


def _row_spec(cols):
    return pl.BlockSpec((BR, cols), lambda r: (r, 0))


def _full_spec(rows, cols):
    return pl.BlockSpec((rows, cols), lambda r: (0, 0))


def kernel(logits, features, edge_index, Wp, bp, W1, b1, W2, b2):
    src = edge_index[0]
    dst = edge_index[1]
    pad_e = EPAD - E
    srcp = jnp.concatenate([src, jnp.full((pad_e,), N, jnp.int32)])
    dstp = jnp.concatenate([dst, jnp.full((pad_e,), N, jnp.int32)])
    srcp = srcp.reshape(TOT_CHUNKS, CHUNK)
    dstp = dstp.reshape(TOT_CHUNKS, CHUNK)

    zeros128 = jnp.zeros((NPAD, HIDDEN), jnp.float32)
    zeros64 = jnp.zeros((NPAD, FEATURE_HIDDEN), jnp.float32)
    zeros1 = jnp.zeros((NPAD,), jnp.float32)
    ones_c = jnp.ones((CHUNK,), jnp.float32)

    logits_p = jnp.zeros((NPAD, NUM_CLASSES), jnp.float32).at[:N].set(logits)
    features_p = jnp.zeros((NPAD, FEATURE_DIM), jnp.float32).at[:N].set(features)

    WpT = Wp.T
    W1aT = W1[:, :NUM_CLASSES].T
    W1bT = W1[:, NUM_CLASSES:].T
    W2Tp = jnp.zeros((HIDDEN, FEATURE_HIDDEN), jnp.float32).at[:, :NUM_CLASSES].set(W2.T)
    b2p = jnp.zeros((FEATURE_HIDDEN,), jnp.float32).at[:NUM_CLASSES].set(b2)

    dpart = _deg_kernel(dstp, ones_c, zeros1)
    d0 = dpart[0].reshape(NPAD, 1)
    d1 = dpart[1].reshape(NPAD, 1)

    g1, dinv = pl.pallas_call(
        _tc1_body,
        grid=(GRID,),
        in_specs=[
            _row_spec(NUM_CLASSES),
            _row_spec(FEATURE_DIM),
            _full_spec(FEATURE_DIM, FEATURE_HIDDEN),
            _full_spec(1, FEATURE_HIDDEN),
            _full_spec(NUM_CLASSES, HIDDEN),
            _full_spec(FEATURE_HIDDEN, HIDDEN),
            _row_spec(1),
            _row_spec(1),
        ],
        out_specs=[_row_spec(HIDDEN), _row_spec(1)],
        out_shape=[
            jax.ShapeDtypeStruct((NPAD, HIDDEN), jnp.float32),
            jax.ShapeDtypeStruct((NPAD, 1), jnp.float32),
        ],
    )(logits_p, features_p, WpT, bp.reshape(1, -1), W1aT, W1bT, d0, d1)

    spart = _spmm128(srcp, dstp, g1, zeros128)

    g2 = pl.pallas_call(
        _tc2_body,
        grid=(GRID,),
        in_specs=[
            _row_spec(HIDDEN),
            _row_spec(HIDDEN),
            _row_spec(HIDDEN),
            _row_spec(1),
            _full_spec(1, HIDDEN),
            _full_spec(HIDDEN, FEATURE_HIDDEN),
        ],
        out_specs=_row_spec(FEATURE_HIDDEN),
        out_shape=jax.ShapeDtypeStruct((NPAD, FEATURE_HIDDEN), jnp.float32),
    )(spart[0], spart[1], g1, dinv, b1.reshape(1, -1), W2Tp)

    tpart = _spmm64(srcp, dstp, g2, zeros64)

    out = pl.pallas_call(
        _tc3_body,
        grid=(GRID,),
        in_specs=[
            _row_spec(FEATURE_HIDDEN),
            _row_spec(FEATURE_HIDDEN),
            _row_spec(FEATURE_HIDDEN),
            _row_spec(1),
            _full_spec(1, FEATURE_HIDDEN),
        ],
        out_specs=_row_spec(FEATURE_HIDDEN),
        out_shape=jax.ShapeDtypeStruct((NPAD, FEATURE_HIDDEN), jnp.float32),
    )(tpart[0], tpart[1], g2, dinv, b2p.reshape(1, -1))

    return out[:N, :NUM_CLASSES]
